# baseline (speedup 1.0000x reference)
import functools

import jax
import jax.numpy as jnp
from jax import lax
from jax.experimental import pallas as pl
from jax.experimental.pallas import tpu as pltpu
from jax.experimental.pallas import tpu_sc as plsc

_B = 32
_D = 64
_GATHER_WORKERS = 4
_ROWS_PER_WORKER = _B // _GATHER_WORKERS


def _sc_gather_body(idx_hbm, table_hbm, out_hbm, idx_v, rows_v, sem):
    c = lax.axis_index("c")
    s = lax.axis_index("s")

    @pl.when((c == 0) & (s < _GATHER_WORKERS))
    def _():
        base = s * _ROWS_PER_WORKER
        pltpu.sync_copy(idx_hbm.at[pl.ds(base, _ROWS_PER_WORKER)], idx_v)
        pltpu.async_copy(table_hbm.at[idx_v], rows_v, sem).wait()
        pltpu.sync_copy(rows_v, out_hbm.at[pl.ds(base, _ROWS_PER_WORKER)])


def _sc_gather(x_init, emb):
    k = pl.kernel(
        _sc_gather_body,
        out_type=jax.ShapeDtypeStruct((_B, _D), jnp.float32),
        mesh=plsc.VectorSubcoreMesh(core_axis_name="c", subcore_axis_name="s"),
        compiler_params=pltpu.CompilerParams(use_tc_tiling_on_sc=False),
        scratch_types=[
            pltpu.VMEM((_ROWS_PER_WORKER,), jnp.int32),
            pltpu.VMEM((_ROWS_PER_WORKER, _D), jnp.float32),
            pltpu.SemaphoreType.DMA,
        ],
    )
    return k(x_init, emb)


def _tc_body(rows_ref, w_ref, out_ref):
    h = rows_ref[...] * jnp.float32(1.5)
    logits = jnp.dot(h, w_ref[...], preferred_element_type=jnp.float32)
    m = jnp.max(logits, axis=1, keepdims=True)
    e = jnp.exp(logits - m)
    out_ref[...] = e / jnp.sum(e, axis=1, keepdims=True)


def kernel(x_init, emb, W, source_p):
    del source_p
    b = x_init.shape[0]
    v = emb.shape[0]
    rows = _sc_gather(x_init.astype(jnp.int32), emb)
    return pl.pallas_call(
        _tc_body,
        out_shape=jax.ShapeDtypeStruct((b, v), jnp.float32),
    )(rows, W)


# speedup vs baseline: 4.1107x; 4.1107x over previous
import jax
import jax.numpy as jnp
from jax.experimental import pallas as pl


def _body(x_ref, emb_ref, w_ref, out_ref):
    b = x_ref.shape[0]
    v, d = emb_ref.shape
    x = x_ref[...]
    cols = jax.lax.broadcasted_iota(jnp.int32, (b, v), 1)
    onehot = (cols == x).astype(jnp.float32)
    h = jnp.dot(onehot, emb_ref[...], preferred_element_type=jnp.float32)
    h = h * jnp.float32(1.5)
    logits = jnp.dot(h, w_ref[...], preferred_element_type=jnp.float32)
    m = jnp.max(logits, axis=1, keepdims=True)
    e = jnp.exp(logits - m)
    out_ref[...] = e / jnp.sum(e, axis=1, keepdims=True)


def kernel(x_init, emb, W, source_p):
    del source_p
    b = x_init.shape[0]
    v = emb.shape[0]
    x2d = x_init.reshape(b, 1).astype(jnp.int32)
    return pl.pallas_call(
        _body,
        out_shape=jax.ShapeDtypeStruct((b, v), jnp.float32),
    )(x2d, emb, W)


# speedup vs baseline: 5.6133x; 1.3655x over previous
import jax
import jax.numpy as jnp
from jax.experimental import pallas as pl
from jax.experimental.pallas import tpu as pltpu


def _body(x_ref, emb_ref, w_ref, out_ref, rows_ref):
    b = rows_ref.shape[0]
    for i in range(b):
        rows_ref[i, :] = emb_ref[x_ref[i], :]
    h = rows_ref[...] * jnp.float32(1.5)
    logits = jnp.dot(h, w_ref[...], preferred_element_type=jnp.float32)
    m = jnp.max(logits, axis=1, keepdims=True)
    e = jnp.exp(logits - m)
    out_ref[...] = e / jnp.sum(e, axis=1, keepdims=True)


def kernel(x_init, emb, W, source_p):
    del source_p
    b = x_init.shape[0]
    v, d = emb.shape
    return pl.pallas_call(
        _body,
        in_specs=[
            pl.BlockSpec(memory_space=pltpu.SMEM),
            pl.BlockSpec(memory_space=pltpu.VMEM),
            pl.BlockSpec(memory_space=pltpu.VMEM),
        ],
        out_specs=pl.BlockSpec(memory_space=pltpu.VMEM),
        scratch_shapes=[pltpu.VMEM((b, d), jnp.float32)],
        out_shape=jax.ShapeDtypeStruct((b, v), jnp.float32),
    )(x_init, emb, W)
